# baseline (speedup 1.0000x reference)
import functools

import jax
import jax.numpy as jnp
from jax import lax
from jax.experimental import pallas as pl
from jax.experimental.pallas import tpu as pltpu
from jax.experimental.pallas import tpu_sc as plsc

_NUM_FIELDS = 26
_VOCAB = 100000
_EMBED = 32
_BATCH = 16384

_INFO = plsc.get_sparse_core_info()
_NC = _INFO.num_cores
_NS = _INFO.num_subcores
_NW = _NC * _NS
_L = 16

_VB = 512
_UPF = _VOCAB // _VB
_TAIL = 128
_REM = _VOCAB - _UPF * _VB - _TAIL
_UNITS = _NUM_FIELDS * _UPF

_BPW = _BATCH // _NW


def _tr_flat_to_flat(src1d, dst1d, n, lanes):
    lanes32 = lanes * _EMBED

    def gbody(g, _):
        row32 = lanes32 + g * _L * _EMBED
        for e in range(_EMBED):
            vals = src1d[pl.ds(e * n + g * _L, _L)]
            plsc.store_scatter(dst1d, [row32 + e], vals)
        return 0

    lax.fori_loop(0, n // _L, gbody, 0)


def _tr_2d_to_flat(src, dst1d, n, lanes):
    rowh = [lanes + h * _L for h in range(_EMBED // _L)]
    zero = jnp.zeros((_L,), jnp.int32)

    def gbody(vg, _):
        for dv in range(_L):
            v = vg * _L + dv
            colv = zero + v
            for h in range(_EMBED // _L):
                vals = plsc.load_gather(src, [rowh[h], colv])
                dst1d[pl.ds(v * _EMBED + h * _L, _L)] = vals
        return 0

    lax.fori_loop(0, n // _L, gbody, 0)


def _tr_Nx32_to_32xN(src, dst, n, lanes):
    rowh = [lanes + h * _L for h in range(_EMBED // _L)]
    zero = jnp.zeros((_L,), jnp.int32)

    def gbody(vg, _):
        for dv in range(_L):
            v = vg * _L + dv
            colv = zero + v
            for h in range(_EMBED // _L):
                vals = plsc.load_gather(src, [colv, rowh[h]])
                plsc.store_scatter(dst, [rowh[h], colv], vals)
        return 0

    lax.fori_loop(0, n // _L, gbody, 0)


@functools.partial(
    pl.kernel,
    mesh=plsc.VectorSubcoreMesh(core_axis_name="c", subcore_axis_name="s"),
    out_type=jax.ShapeDtypeStruct((_NUM_FIELDS * _VOCAB * _EMBED,),
                                  jnp.float32),
    scratch_types=[
        pltpu.VMEM((_EMBED, _VB + 1), jnp.float32),
        pltpu.VMEM((_EMBED, _VB + 1), jnp.float32),
        pltpu.VMEM((_VB * _EMBED,), jnp.float32),
        pltpu.VMEM((_VB * _EMBED,), jnp.float32),
        pltpu.VMEM((_EMBED * _TAIL,), jnp.float32),
        pltpu.VMEM((_TAIL * _EMBED,), jnp.float32),
        pltpu.VMEM((_REM * _EMBED,), jnp.float32),
        pltpu.SemaphoreType.DMA,
        pltpu.SemaphoreType.DMA,
    ],
    compiler_params=pltpu.CompilerParams(
        use_tc_tiling_on_sc=True, needs_layout_passes=False
    ),
)
def _retile(tt_hbm, tail_hbm, flat_hbm, inb0, inb1, outb0, outb1, tinb,
            toutb, remb, isem, osem):
    w = lax.axis_index("s") * _NC + lax.axis_index("c")
    lanes = lax.iota(jnp.int32, _L)
    n_units = (_UNITS - w + _NW - 1) // _NW

    def unit_fv(u):
        f = u // _UPF
        vb = (u % _UPF) * _VB
        return f, vb

    def stage(u, inb):
        f, vb = unit_fv(u)
        pltpu.async_copy(
            tt_hbm.at[f, slice(None), pl.ds(vb, _VB)],
            inb.at[slice(None), pl.ds(0, _VB)], isem)

    def put(u, outb):
        f, vb = unit_fv(u)
        pltpu.async_copy(
            outb,
            flat_hbm.at[pl.ds((f * _VOCAB + vb) * _EMBED, _VB * _EMBED)],
            osem)

    stage(w, inb0)

    def body(k, _):
        u = w + k * _NW
        b = k % 2
        pltpu.make_async_copy(
            tt_hbm.at[0, slice(None), pl.ds(0, _VB)],
            inb0.at[slice(None), pl.ds(0, _VB)], isem).wait()

        @pl.when(jnp.logical_and(k + 1 < n_units, b == 0))
        def _():
            stage(u + _NW, inb1)

        @pl.when(jnp.logical_and(k + 1 < n_units, b == 1))
        def _():
            stage(u + _NW, inb0)

        @pl.when(k >= 2)
        def _():
            pltpu.make_async_copy(
                outb0, flat_hbm.at[pl.ds(0, _VB * _EMBED)], osem).wait()

        @pl.when(b == 0)
        def _():
            _tr_2d_to_flat(inb0, outb0, _VB, lanes)
            put(u, outb0)

        @pl.when(b == 1)
        def _():
            _tr_2d_to_flat(inb1, outb1, _VB, lanes)
            put(u, outb1)

        return 0

    lax.fori_loop(0, n_units, body, 0)

    @pl.when(n_units >= 1)
    def _():
        pltpu.make_async_copy(
            outb0, flat_hbm.at[pl.ds(0, _VB * _EMBED)], osem).wait()

    @pl.when(n_units >= 2)
    def _():
        pltpu.make_async_copy(
            outb0, flat_hbm.at[pl.ds(0, _VB * _EMBED)], osem).wait()

    @pl.when(w < _NUM_FIELDS)
    def _():
        vb = _UPF * _VB
        for e in range(_EMBED):
            pltpu.sync_copy(tt_hbm.at[w, e, pl.ds(vb, _TAIL)],
                            tinb.at[pl.ds(e * _TAIL, _TAIL)])
        _tr_flat_to_flat(tinb, toutb, _TAIL, lanes)
        pltpu.sync_copy(
            toutb,
            flat_hbm.at[pl.ds((w * _VOCAB + vb) * _EMBED, _TAIL * _EMBED)])
        pltpu.sync_copy(tail_hbm.at[pl.ds(w * _REM * _EMBED, _REM * _EMBED)],
                        remb)
        pltpu.sync_copy(
            remb,
            flat_hbm.at[pl.ds((w * _VOCAB + vb + _TAIL) * _EMBED,
                              _REM * _EMBED)])


@functools.partial(
    pl.kernel,
    mesh=plsc.VectorSubcoreMesh(core_axis_name="c", subcore_axis_name="s"),
    out_type=tuple(
        jax.ShapeDtypeStruct((_EMBED, _BATCH), jnp.float32)
        for _ in range(_NUM_FIELDS)
    ),
    scratch_types=[
        pltpu.VMEM((_BPW,), jnp.int32),
        pltpu.VMEM((_NUM_FIELDS, _BPW), jnp.int32),
        pltpu.VMEM((_BPW, _EMBED), jnp.float32),
        pltpu.VMEM((_BPW, _EMBED), jnp.float32),
        pltpu.VMEM((_EMBED, _BPW + 1), jnp.float32),
        pltpu.VMEM((_EMBED, _BPW + 1), jnp.float32),
        pltpu.SemaphoreType.DMA,
        pltpu.SemaphoreType.DMA,
    ],
    compiler_params=pltpu.CompilerParams(
        use_tc_tiling_on_sc=False, needs_layout_passes=False
    ),
)
def _gather(flat_hbm, x_t_hbm, *refs):
    outs = refs[:_NUM_FIELDS]
    idx_v, idx_all, rows0, rows1, trb0, trb1, gsem, osem = refs[_NUM_FIELDS:]
    w = lax.axis_index("s") * _NC + lax.axis_index("c")
    base = w * _BPW
    lanes = lax.iota(jnp.int32, _L)

    pltpu.sync_copy(x_t_hbm.at[slice(None), pl.ds(base, _BPW)], idx_all)

    def load_idx(f):
        frow = jnp.zeros((_L,), jnp.int32) + f
        off = jnp.zeros((_L,), jnp.int32) + f * _VOCAB
        for g in range(_BPW // _L):
            vals = plsc.load_gather(idx_all, [frow, lanes + g * _L])
            idx_v[pl.ds(g * _L, _L)] = vals + off

    def write_out(f, trb):
        for ff in range(_NUM_FIELDS):
            @pl.when(f == ff)
            def _(ff=ff):
                pltpu.async_copy(
                    trb, outs[ff].at[slice(None), pl.ds(base, _BPW)], osem)

    load_idx(0)
    pltpu.async_copy(flat_hbm.at[idx_v], rows0, gsem)

    def body(f, _):
        b = f % 2
        pltpu.make_async_copy(flat_hbm.at[idx_v], rows0, gsem).wait()

        @pl.when(f + 1 < _NUM_FIELDS)
        def _():
            load_idx(f + 1)

            @pl.when(b == 0)
            def _():
                pltpu.async_copy(flat_hbm.at[idx_v], rows1, gsem)

            @pl.when(b == 1)
            def _():
                pltpu.async_copy(flat_hbm.at[idx_v], rows0, gsem)

        @pl.when(f >= 2)
        def _():
            pltpu.make_async_copy(
                trb0.at[slice(None), pl.ds(0, _BPW)],
                outs[0].at[slice(None), pl.ds(base, _BPW)],
                osem).wait()

        @pl.when(b == 0)
        def _():
            _tr_Nx32_to_32xN(rows0, trb0, _BPW, lanes)
            write_out(f, trb0.at[slice(None), pl.ds(0, _BPW)])

        @pl.when(b == 1)
        def _():
            _tr_Nx32_to_32xN(rows1, trb1, _BPW, lanes)
            write_out(f, trb1.at[slice(None), pl.ds(0, _BPW)])

        return 0

    lax.fori_loop(0, _NUM_FIELDS, body, 0)

    pltpu.make_async_copy(
        trb0.at[slice(None), pl.ds(0, _BPW)],
        outs[0].at[slice(None), pl.ds(base, _BPW)], osem).wait()
    pltpu.make_async_copy(
        trb0.at[slice(None), pl.ds(0, _BPW)],
        outs[0].at[slice(None), pl.ds(base, _BPW)], osem).wait()


def kernel(tables, x):
    table_t = tables.transpose(0, 2, 1)
    tail_rm = tables[:, _UPF * _VB + _TAIL:, :].reshape(-1)
    flat = _retile(table_t, tail_rm)
    flat2 = flat.reshape(_NUM_FIELDS * _VOCAB, _EMBED)
    x_t = x.T
    outs_t = _gather(flat2, x_t)
    return tuple(o.T for o in outs_t)


# speedup vs baseline: 2.0164x; 2.0164x over previous
import functools

import jax
import jax.numpy as jnp
from jax import lax
from jax.experimental import pallas as pl
from jax.experimental.pallas import tpu as pltpu
from jax.experimental.pallas import tpu_sc as plsc

_NUM_FIELDS = 26
_VOCAB = 100000
_EMBED = 32
_BATCH = 16384

_INFO = plsc.get_sparse_core_info()
_NC = _INFO.num_cores
_NS = _INFO.num_subcores
_NW = _NC * _NS


@functools.partial(
    pl.kernel,
    mesh=plsc.VectorSubcoreMesh(core_axis_name="c", subcore_axis_name="s"),
    out_type=tuple(
        jax.ShapeDtypeStruct((_EMBED, _BATCH), jnp.float32)
        for _ in range(_NUM_FIELDS)
    ),
    scratch_types=[
        pltpu.VMEM((2, _BATCH), jnp.int32),
        pltpu.VMEM((2, _BATCH), jnp.float32),
        pltpu.SemaphoreType.DMA,
        pltpu.SemaphoreType.DMA,
        pltpu.SemaphoreType.DMA,
    ],
    compiler_params=pltpu.CompilerParams(
        use_tc_tiling_on_sc=False, needs_layout_passes=False
    ),
)
def _embed_all(table_t_hbm, x_t_hbm, *refs):
    outs = refs[:_NUM_FIELDS]
    idx_v, val_v, isem, gsem, osem = refs[_NUM_FIELDS:]
    e = lax.axis_index("s") * _NC + lax.axis_index("c")

    pltpu.async_copy(x_t_hbm.at[0], idx_v.at[0], isem).wait()
    pltpu.async_copy(x_t_hbm.at[1], idx_v.at[1], isem)
    pltpu.async_copy(table_t_hbm.at[e].at[idx_v.at[0]], val_v.at[0], gsem)
    for f in range(_NUM_FIELDS):
        b = f % 2
        nb = (f + 1) % 2
        pltpu.make_async_copy(table_t_hbm.at[0].at[idx_v.at[b]],
                              val_v.at[b], gsem).wait()
        if f + 1 < _NUM_FIELDS:
            pltpu.make_async_copy(x_t_hbm.at[0], idx_v.at[0], isem).wait()
            if f >= 1:
                pltpu.make_async_copy(val_v.at[0], outs[0].at[e], osem).wait()
            pltpu.async_copy(
                table_t_hbm.at[(f + 1) * _EMBED + e].at[idx_v.at[nb]],
                val_v.at[nb], gsem)
            if f + 2 < _NUM_FIELDS:
                pltpu.async_copy(x_t_hbm.at[f + 2], idx_v.at[b], isem)
        pltpu.async_copy(val_v.at[b], outs[f].at[e], osem)
    pltpu.make_async_copy(val_v.at[0], outs[0].at[e], osem).wait()
    pltpu.make_async_copy(val_v.at[0], outs[0].at[e], osem).wait()


def kernel(tables, x):
    table_t = tables.transpose(0, 2, 1).reshape(_NUM_FIELDS * _EMBED, _VOCAB)
    x_t = x.T
    outs_t = _embed_all(table_t, x_t)
    return tuple(o.T for o in outs_t)
